# Initial kernel scaffold; baseline (speedup 1.0000x reference)
import functools

import jax
import jax.numpy as jnp
from jax import lax
from jax.experimental import pallas as pl
from jax.experimental.pallas import tpu as pltpu
from jax.experimental.pallas import tpu_sc as plsc

DIM = 64
B = 16384
NEG = 20
J = NEG + 1
NC = 2
NS = 16
NW = NC * NS
BPW = B // NW
QCH = 128
QN = BPW // QCH
LANES = 16


def _sc_mesh():
    return plsc.VectorSubcoreMesh(core_axis_name="c", subcore_axis_name="s")


@functools.partial(
    pl.kernel,
    mesh=_sc_mesh(),
    out_type=jax.ShapeDtypeStruct((NW, J, BPW), jnp.float32),
    scratch_types=[
        pltpu.VMEM((QN, QCH), jnp.int32),
        pltpu.VMEM((J, QN, QCH), jnp.int32),
        pltpu.VMEM((BPW, DIM), jnp.float32),
        pltpu.VMEM((BPW, DIM), jnp.float32),
        pltpu.VMEM((J, BPW), jnp.float32),
        pltpu.SemaphoreType.DMA,
    ],
)
def _sc_scores(tidx_hbm, cn_hbm, wt_hbm, wc_hbm, out_hbm,
               tidx_v, cidx_v, t_rows, r_buf, scores_v, sem):
    wid = lax.axis_index("s") * NC + lax.axis_index("c")

    pltpu.sync_copy(tidx_hbm.at[wid], tidx_v)
    pltpu.sync_copy(cn_hbm.at[:, wid], cidx_v)

    for q in range(QN):
        pltpu.async_copy(wt_hbm.at[tidx_v.at[q]],
                         t_rows.at[pl.ds(q * QCH, QCH)], sem).wait()

    def dot_rows(j, i):
        acc = t_rows[i, pl.ds(0, LANES)] * r_buf[i, pl.ds(0, LANES)]
        for d in range(1, DIM // LANES):
            acc = acc + (t_rows[i, pl.ds(d * LANES, LANES)]
                         * r_buf[i, pl.ds(d * LANES, LANES)])
        scores_v[j, i] = jnp.sum(acc)

    def j_body(j, carry):
        for q in range(QN):
            pltpu.async_copy(wc_hbm.at[cidx_v.at[j, q]],
                             r_buf.at[pl.ds(q * QCH, QCH)], sem).wait()

        def i_body(i, c):
            dot_rows(j, i)
            return c

        lax.fori_loop(0, BPW, i_body, carry)
        return carry

    lax.fori_loop(0, J, j_body, 0)
    pltpu.sync_copy(scores_v, out_hbm.at[wid])


def _tc_loss_body(s_ref, o_ref):
    s = s_ref[...]
    row = lax.broadcasted_iota(jnp.int32, s.shape, 0)
    x = jnp.where(row % J == 0, -s, s)
    sp = jnp.maximum(x, 0.0) + jnp.log1p(jnp.exp(-jnp.abs(x)))
    o_ref[0, 0] = jnp.sum(sp) * (1.0 / B)


def kernel(target, context, negatives, W_target, W_context):
    tgt = target.astype(jnp.int32)
    cn = jnp.concatenate(
        [context.astype(jnp.int32)[None, :], negatives.astype(jnp.int32).T],
        axis=0)
    tidx = tgt.reshape(NW, QN, QCH)
    cnidx = cn.reshape(J, NW, QN, QCH)

    scores = _sc_scores(tidx, cnidx, W_target, W_context)

    loss = pl.pallas_call(
        _tc_loss_body,
        out_shape=jax.ShapeDtypeStruct((1, 1), jnp.float32),
    )(scores.reshape(NW * J, BPW))
    return loss[0, 0]


# speedup vs baseline: 3.6243x; 3.6243x over previous
import functools

import jax
import jax.numpy as jnp
from jax import lax
from jax.experimental import pallas as pl
from jax.experimental.pallas import tpu as pltpu
from jax.experimental.pallas import tpu_sc as plsc

DIM = 64
B = 16384
NEG = 20
J = NEG + 1
NC = 2
NS = 16
NW = NC * NS
BPW = B // NW
QCH = 128
QN = BPW // QCH
LANES = 16


@functools.partial(
    pl.kernel,
    mesh=plsc.VectorSubcoreMesh(core_axis_name="c", subcore_axis_name="s"),
    compiler_params=pltpu.CompilerParams(use_tc_tiling_on_sc=False),
    out_type=jax.ShapeDtypeStruct((NW, J, BPW, LANES), jnp.float32),
    scratch_types=[
        pltpu.VMEM((QN, QCH), jnp.int32),
        pltpu.VMEM((J, QN, QCH), jnp.int32),
        pltpu.VMEM((BPW, DIM), jnp.float32),
        pltpu.VMEM((BPW, DIM), jnp.float32),
        pltpu.VMEM((BPW, LANES), jnp.float32),
        pltpu.SemaphoreType.DMA,
    ],
)
def _sc_partials(tidx_hbm, cn_hbm, wt_hbm, wc_hbm, out_hbm,
                 tidx_v, cidx_v, t_rows, r_buf, psum_v, sem):
    wid = lax.axis_index("s") * NC + lax.axis_index("c")

    pltpu.sync_copy(tidx_hbm.at[wid], tidx_v)
    pltpu.sync_copy(cn_hbm.at[:, wid], cidx_v)

    for q in range(QN):
        pltpu.async_copy(wt_hbm.at[tidx_v.at[q]],
                         t_rows.at[pl.ds(q * QCH, QCH)], sem).wait()

    def j_body(j, carry):
        for q in range(QN):
            pltpu.async_copy(wc_hbm.at[cidx_v.at[j, q]],
                             r_buf.at[pl.ds(q * QCH, QCH)], sem).wait()

        def i_body(i, c):
            acc = t_rows[i, pl.ds(0, LANES)] * r_buf[i, pl.ds(0, LANES)]
            for d in range(1, DIM // LANES):
                acc = acc + (t_rows[i, pl.ds(d * LANES, LANES)]
                             * r_buf[i, pl.ds(d * LANES, LANES)])
            psum_v[i] = acc
            return c

        lax.fori_loop(0, BPW, i_body, carry, unroll=4)
        pltpu.sync_copy(psum_v, out_hbm.at[wid, j])
        return carry

    lax.fori_loop(0, J, j_body, 0)


ROWS = NW * J * BPW
RB = 4096
GRID = ROWS // RB


def _tc_loss_body(s_ref, o_ref, acc_ref):
    g = pl.program_id(0)

    @pl.when(g == 0)
    def _init():
        acc_ref[0] = 0.0

    s = jnp.sum(s_ref[...], axis=1, keepdims=True)
    row = g * RB + lax.broadcasted_iota(jnp.int32, (RB, 1), 0)
    x = jnp.where((row // BPW) % J == 0, -s, s)
    sp = jnp.maximum(x, 0.0) + jnp.log1p(jnp.exp(-jnp.abs(x)))
    acc_ref[0] = acc_ref[0] + jnp.sum(sp)

    @pl.when(g == GRID - 1)
    def _done():
        o_ref[0, 0] = acc_ref[0] * (1.0 / B)


def kernel(target, context, negatives, W_target, W_context):
    tgt = target.astype(jnp.int32)
    cn = jnp.concatenate(
        [context.astype(jnp.int32)[None, :], negatives.astype(jnp.int32).T],
        axis=0)
    tidx = tgt.reshape(NW, QN, QCH)
    cnidx = cn.reshape(J, NW, QN, QCH)

    partials = _sc_partials(tidx, cnidx, W_target, W_context)

    loss = pl.pallas_call(
        _tc_loss_body,
        grid=(GRID,),
        in_specs=[pl.BlockSpec((RB, LANES), lambda g: (g, 0))],
        out_shape=jax.ShapeDtypeStruct((1, 1), jnp.float32),
        out_specs=pl.BlockSpec((1, 1), lambda g: (0, 0),
                               memory_space=pltpu.SMEM),
        scratch_shapes=[pltpu.SMEM((1,), jnp.float32)],
    )(partials.reshape(ROWS, LANES))
    return loss[0, 0]
